# baseline (speedup 1.0000x reference)
import functools

import jax
import jax.numpy as jnp
from jax import lax
from jax.experimental import pallas as pl
from jax.experimental.pallas import tpu as pltpu
from jax.experimental.pallas import tpu_sc as plsc

N = 10000
E = 320000
CIN = 128
COUT = 128
R = 8

NC = 2
NS = 16
NW = NC * NS
K = 128
NCH = 80
NCH_H = NCH // 2
EPAD = NW * NCH * K
NTRASH = 16
N_PAD = N + NTRASH
ROWS_PER_SUB = 640

BN = 200
NB = N // BN
EB = E // NB // 128


def _tc_transform_body(x_ref, w_ref, src_ref, et_ref, xw_ref, fidx_ref):
    xb = x_ref[...]
    for r in range(R):
        xw_ref[:, r, :] = jnp.dot(xb, w_ref[r], preferred_element_type=jnp.float32)
    fidx_ref[...] = src_ref[...] * R + et_ref[...]


_tc_transform = pl.pallas_call(
    _tc_transform_body,
    grid=(NB,),
    in_specs=[
        pl.BlockSpec((BN, CIN), lambda i: (i, 0)),
        pl.BlockSpec((R, CIN, COUT), lambda i: (0, 0, 0)),
        pl.BlockSpec((1, EB, 128), lambda i: (i, 0, 0)),
        pl.BlockSpec((1, EB, 128), lambda i: (i, 0, 0)),
    ],
    out_specs=[
        pl.BlockSpec((BN, R, COUT), lambda i: (i, 0, 0)),
        pl.BlockSpec((1, EB, 128), lambda i: (i, 0, 0)),
    ],
    out_shape=[
        jax.ShapeDtypeStruct((N, R, COUT), jnp.float32),
        jax.ShapeDtypeStruct((NB, EB, 128), jnp.int32),
    ],
)


def _sc_body(fidx_hbm, dst_hbm, xw_hbm, zeros_hbm, out_hbm,
             fidx_v, dst_v, rows0_v, rows1_v, acc_sh, sem0, sem1, ssem0, ssem1):
    c = lax.axis_index("c")
    s = lax.axis_index("s")
    wid = s * NC + c

    row0 = pl.multiple_of(
        jnp.where(s == NS - 1, N_PAD - ROWS_PER_SUB, s * ROWS_PER_SUB), 8)
    pltpu.sync_copy(zeros_hbm.at[pl.ds(row0, ROWS_PER_SUB)],
                    acc_sh.at[pl.ds(row0, ROWS_PER_SUB)])
    plsc.subcore_barrier()

    for h in range(NCH // NCH_H):
        pltpu.sync_copy(fidx_hbm.at[wid, pl.ds(h * NCH_H, NCH_H)], fidx_v)
        pltpu.sync_copy(dst_hbm.at[wid, pl.ds(h * NCH_H, NCH_H)], dst_v)
        pltpu.async_copy(xw_hbm.at[fidx_v.at[0]], rows0_v, sem0)
        pltpu.async_copy(xw_hbm.at[fidx_v.at[1]], rows1_v, sem1)

        def pair(p, carry):
            j0 = p * 2
            pltpu.make_async_copy(xw_hbm.at[fidx_v.at[0]], rows0_v, sem0).wait()
            pltpu.async_copy(rows0_v, acc_sh.at[dst_v.at[j0]], ssem0, add=True)
            pltpu.make_async_copy(xw_hbm.at[fidx_v.at[0]], rows1_v, sem1).wait()
            pltpu.async_copy(rows1_v, acc_sh.at[dst_v.at[j0 + 1]], ssem1, add=True)

            pltpu.make_async_copy(rows0_v, acc_sh.at[dst_v.at[j0]], ssem0).wait()

            @pl.when(j0 + 2 < NCH_H)
            def _g0():
                pltpu.async_copy(xw_hbm.at[fidx_v.at[j0 + 2]], rows0_v, sem0)

            pltpu.make_async_copy(rows1_v, acc_sh.at[dst_v.at[j0 + 1]], ssem1).wait()

            @pl.when(j0 + 3 < NCH_H)
            def _g1():
                pltpu.async_copy(xw_hbm.at[fidx_v.at[j0 + 3]], rows1_v, sem1)

            return carry

        lax.fori_loop(0, NCH_H // 2, pair, 0)
    plsc.subcore_barrier()
    pltpu.sync_copy(acc_sh.at[pl.ds(row0, ROWS_PER_SUB)],
                    out_hbm.at[c, pl.ds(row0, ROWS_PER_SUB)])


@functools.cache
def _sc_scatter():
    return functools.partial(
        pl.kernel,
        mesh=plsc.VectorSubcoreMesh(core_axis_name="c", subcore_axis_name="s"),
        out_type=jax.ShapeDtypeStruct((NC, N_PAD, COUT), jnp.float32),
        scratch_types=[
            pltpu.VMEM((NCH_H, K), jnp.int32),
            pltpu.VMEM((NCH_H, K), jnp.int32),
            pltpu.VMEM((K, COUT), jnp.float32),
            pltpu.VMEM((K, COUT), jnp.float32),
            pltpu.VMEM_SHARED((N_PAD, COUT), jnp.float32),
            pltpu.SemaphoreType.DMA,
            pltpu.SemaphoreType.DMA,
            pltpu.SemaphoreType.DMA,
            pltpu.SemaphoreType.DMA,
        ],
    )(_sc_body)


def _tc_final_body(x_ref, wr_ref, b_ref, p_ref, o_ref):
    acc = jnp.dot(x_ref[...], wr_ref[...], preferred_element_type=jnp.float32)
    acc = acc + p_ref[0] + p_ref[1] + b_ref[...]
    o_ref[...] = jnp.maximum(acc, 0.0)


_tc_final = pl.pallas_call(
    _tc_final_body,
    grid=(NB,),
    in_specs=[
        pl.BlockSpec((BN, CIN), lambda i: (i, 0)),
        pl.BlockSpec((CIN, COUT), lambda i: (0, 0)),
        pl.BlockSpec((1, COUT), lambda i: (0, 0)),
        pl.BlockSpec((NC, BN, COUT), lambda i: (0, i, 0)),
    ],
    out_specs=pl.BlockSpec((BN, COUT), lambda i: (i, 0)),
    out_shape=jax.ShapeDtypeStruct((N, COUT), jnp.float32),
)


def kernel(x, edge_index, edge_type, W, W_root, b):
    src = edge_index[0].astype(jnp.int32).reshape(NB, EB, 128)
    et = edge_type.astype(jnp.int32).reshape(NB, EB, 128)
    xw, fidx = _tc_transform(x, W, src, et)
    npad = EPAD - E
    pad_i = jnp.arange(npad, dtype=jnp.int32)
    fidx3 = jnp.concatenate(
        [fidx.reshape(E), (pad_i * 64) % (N * R)]).reshape(NW, NCH, K)
    dst3 = jnp.concatenate(
        [edge_index[1].astype(jnp.int32), N + (pad_i % NTRASH)]).reshape(NW, NCH, K)
    zeros = jnp.zeros((N_PAD, COUT), jnp.float32)
    partials = _sc_scatter()(fidx3, dst3, xw.reshape(N * R, COUT), zeros)
    return _tc_final(x, W_root, b.reshape(1, COUT), partials)


# speedup vs baseline: 1.1872x; 1.1872x over previous
import functools

import jax
import jax.numpy as jnp
from jax import lax
from jax.experimental import pallas as pl
from jax.experimental.pallas import tpu as pltpu
from jax.experimental.pallas import tpu_sc as plsc

N = 10000
E = 320000
CIN = 128
COUT = 128
R = 8

NC = 2
NS = 16
NW = NC * NS
K = 128
NCH = 80
NCH_H = NCH // 2
EPAD = NW * NCH * K
NTRASH = 16
N_PAD = N + NTRASH
ROWS_PER_SUB = 640

BN = 200
NB = N // BN
EB = E // NB // 128


def _tc_transform_body(x_ref, w_ref, src_ref, et_ref, xw_ref, fidx_ref):
    xb = x_ref[...]
    for r in range(R):
        xw_ref[:, r, :] = jnp.dot(xb, w_ref[r], preferred_element_type=jnp.float32)
    fidx_ref[...] = src_ref[...] * R + et_ref[...]


_tc_transform = pl.pallas_call(
    _tc_transform_body,
    grid=(NB,),
    in_specs=[
        pl.BlockSpec((BN, CIN), lambda i: (i, 0)),
        pl.BlockSpec((R, CIN, COUT), lambda i: (0, 0, 0)),
        pl.BlockSpec((1, EB, 128), lambda i: (i, 0, 0)),
        pl.BlockSpec((1, EB, 128), lambda i: (i, 0, 0)),
    ],
    out_specs=[
        pl.BlockSpec((BN, R, COUT), lambda i: (i, 0, 0)),
        pl.BlockSpec((1, EB, 128), lambda i: (i, 0, 0)),
    ],
    out_shape=[
        jax.ShapeDtypeStruct((N, R, COUT), jnp.float32),
        jax.ShapeDtypeStruct((NB, EB, 128), jnp.int32),
    ],
)


def _sc_body(fidx_hbm, dst_hbm, xw_hbm, out_hbm,
             fidx_v, dst_v, rows0_v, rows1_v, acc_sh, sem0, sem1):
    c = lax.axis_index("c")
    s = lax.axis_index("s")
    wid = s * NC + c

    row0 = pl.multiple_of(
        jnp.where(s == NS - 1, N_PAD - ROWS_PER_SUB, s * ROWS_PER_SUB), 8)

    zv = jnp.zeros((16,), jnp.float32)

    def zrow(i, carry):
        for j in range(COUT // 16):
            rows0_v[i, pl.ds(j * 16, 16)] = zv
        return carry

    lax.fori_loop(0, K, zrow, 0)
    for t in range(ROWS_PER_SUB // K):
        pltpu.sync_copy(rows0_v, acc_sh.at[pl.ds(row0 + t * K, K)])
    plsc.subcore_barrier()

    for h in range(NCH // NCH_H):
        pltpu.sync_copy(fidx_hbm.at[wid, pl.ds(h * NCH_H, NCH_H)], fidx_v)
        pltpu.sync_copy(dst_hbm.at[wid, pl.ds(h * NCH_H, NCH_H)], dst_v)
        pltpu.async_copy(xw_hbm.at[fidx_v.at[0]], rows0_v, sem0)
        pltpu.async_copy(xw_hbm.at[fidx_v.at[1]], rows1_v, sem1)

        def pair(p, carry):
            j0 = p * 2
            pltpu.make_async_copy(xw_hbm.at[fidx_v.at[0]], rows0_v, sem0).wait()
            pltpu.sync_copy(rows0_v, acc_sh.at[dst_v.at[j0]], add=True)

            @pl.when(j0 + 2 < NCH_H)
            def _g0():
                pltpu.async_copy(xw_hbm.at[fidx_v.at[j0 + 2]], rows0_v, sem0)

            pltpu.make_async_copy(xw_hbm.at[fidx_v.at[0]], rows1_v, sem1).wait()
            pltpu.sync_copy(rows1_v, acc_sh.at[dst_v.at[j0 + 1]], add=True)

            @pl.when(j0 + 3 < NCH_H)
            def _g1():
                pltpu.async_copy(xw_hbm.at[fidx_v.at[j0 + 3]], rows1_v, sem1)

            return carry

        lax.fori_loop(0, NCH_H // 2, pair, 0)
    plsc.subcore_barrier()
    pltpu.sync_copy(acc_sh.at[pl.ds(row0, ROWS_PER_SUB)],
                    out_hbm.at[c, pl.ds(row0, ROWS_PER_SUB)])


@functools.cache
def _sc_scatter():
    return functools.partial(
        pl.kernel,
        mesh=plsc.VectorSubcoreMesh(core_axis_name="c", subcore_axis_name="s"),
        out_type=jax.ShapeDtypeStruct((NC, N_PAD, COUT), jnp.float32),
        scratch_types=[
            pltpu.VMEM((NCH_H, K), jnp.int32),
            pltpu.VMEM((NCH_H, K), jnp.int32),
            pltpu.VMEM((K, COUT), jnp.float32),
            pltpu.VMEM((K, COUT), jnp.float32),
            pltpu.VMEM_SHARED((N_PAD, COUT), jnp.float32),
            pltpu.SemaphoreType.DMA,
            pltpu.SemaphoreType.DMA,
        ],
    )(_sc_body)


def _tc_final_body(x_ref, wr_ref, b_ref, p_ref, o_ref):
    acc = jnp.dot(x_ref[...], wr_ref[...], preferred_element_type=jnp.float32)
    acc = acc + p_ref[0] + p_ref[1] + b_ref[...]
    o_ref[...] = jnp.maximum(acc, 0.0)


_tc_final = pl.pallas_call(
    _tc_final_body,
    grid=(NB,),
    in_specs=[
        pl.BlockSpec((BN, CIN), lambda i: (i, 0)),
        pl.BlockSpec((CIN, COUT), lambda i: (0, 0)),
        pl.BlockSpec((1, COUT), lambda i: (0, 0)),
        pl.BlockSpec((NC, BN, COUT), lambda i: (0, i, 0)),
    ],
    out_specs=pl.BlockSpec((BN, COUT), lambda i: (i, 0)),
    out_shape=jax.ShapeDtypeStruct((N, COUT), jnp.float32),
)


def kernel(x, edge_index, edge_type, W, W_root, b):
    src = edge_index[0].astype(jnp.int32).reshape(NB, EB, 128)
    et = edge_type.astype(jnp.int32).reshape(NB, EB, 128)
    xw, fidx = _tc_transform(x, W, src, et)
    npad = EPAD - E
    pad_i = jnp.arange(npad, dtype=jnp.int32)
    fidx3 = jnp.concatenate(
        [fidx.reshape(E), (pad_i * 64) % (N * R)]).reshape(NW, NCH, K)
    dst3 = jnp.concatenate(
        [edge_index[1].astype(jnp.int32), N + (pad_i % NTRASH)]).reshape(NW, NCH, K)
    partials = _sc_scatter()(fidx3, dst3, xw.reshape(N * R, COUT))
    return _tc_final(x, W_root, b.reshape(1, COUT), partials)


# speedup vs baseline: 1.3763x; 1.1593x over previous
import functools

import jax
import jax.numpy as jnp
from jax import lax
from jax.experimental import pallas as pl
from jax.experimental.pallas import tpu as pltpu
from jax.experimental.pallas import tpu_sc as plsc

N = 10000
E = 320000
CIN = 128
COUT = 128
R = 8

NC = 2
NS = 16
NW = NC * NS
K = 128
NCH = 80
NCH_H = NCH // 2
EPAD = NW * NCH * K
NTRASH = 16
N_PAD = N + NTRASH
ROWS_PER_SUB = 640

BN = 400
BNF = 200
NB = N // BN
EB = E // NB // 128
EB2 = E // (N // BN) // 128


def _tc_transform_body(x_ref, w_ref, src_ref, et_ref, xw_ref, fidx_ref):
    xb = x_ref[...]
    for r in range(R):
        xw_ref[:, r, :] = jnp.dot(xb, w_ref[r], preferred_element_type=jnp.float32)
    fidx_ref[...] = src_ref[...] * R + et_ref[...]


_tc_transform = pl.pallas_call(
    _tc_transform_body,
    grid=(N // BN,),
    in_specs=[
        pl.BlockSpec((BN, CIN), lambda i: (i, 0)),
        pl.BlockSpec((R, CIN, COUT), lambda i: (0, 0, 0)),
        pl.BlockSpec((1, EB2, 128), lambda i: (i, 0, 0)),
        pl.BlockSpec((1, EB2, 128), lambda i: (i, 0, 0)),
    ],
    out_specs=[
        pl.BlockSpec((BN, R, COUT), lambda i: (i, 0, 0)),
        pl.BlockSpec((1, EB2, 128), lambda i: (i, 0, 0)),
    ],
    out_shape=[
        jax.ShapeDtypeStruct((N, R, COUT), jnp.float32),
        jax.ShapeDtypeStruct((N // BN, EB2, 128), jnp.int32),
    ],
)


def _sc_body(fd_hbm, xw_hbm, out_hbm,
             fidx_v, dst_v, rows0_v, rows1_v, acc_sh, sem0, sem1):
    c = lax.axis_index("c")
    s = lax.axis_index("s")
    wid = s * NC + c

    row0 = pl.multiple_of(
        jnp.where(s == NS - 1, N_PAD - ROWS_PER_SUB, s * ROWS_PER_SUB), 8)

    zv = jnp.zeros((16,), jnp.float32)

    def zrow(i, carry):
        for j in range(COUT // 16):
            rows0_v[i, pl.ds(j * 16, 16)] = zv
        return carry

    lax.fori_loop(0, K, zrow, 0)
    for t in range(ROWS_PER_SUB // K):
        pltpu.sync_copy(rows0_v, acc_sh.at[pl.ds(row0 + t * K, K)])
    plsc.subcore_barrier()

    for h in range(NCH // NCH_H):
        pltpu.sync_copy(fd_hbm.at[0, wid, pl.ds(h * NCH_H, NCH_H)], fidx_v)
        pltpu.sync_copy(fd_hbm.at[1, wid, pl.ds(h * NCH_H, NCH_H)], dst_v)
        pltpu.async_copy(xw_hbm.at[fidx_v.at[0]], rows0_v, sem0)
        pltpu.async_copy(xw_hbm.at[fidx_v.at[1]], rows1_v, sem1)

        def pair(p, carry):
            j0 = p * 2
            pltpu.make_async_copy(xw_hbm.at[fidx_v.at[0]], rows0_v, sem0).wait()
            pltpu.sync_copy(rows0_v, acc_sh.at[dst_v.at[j0]], add=True)

            @pl.when(j0 + 2 < NCH_H)
            def _g0():
                pltpu.async_copy(xw_hbm.at[fidx_v.at[j0 + 2]], rows0_v, sem0)

            pltpu.make_async_copy(xw_hbm.at[fidx_v.at[0]], rows1_v, sem1).wait()
            pltpu.sync_copy(rows1_v, acc_sh.at[dst_v.at[j0 + 1]], add=True)

            @pl.when(j0 + 3 < NCH_H)
            def _g1():
                pltpu.async_copy(xw_hbm.at[fidx_v.at[j0 + 3]], rows1_v, sem1)

            return carry

        lax.fori_loop(0, NCH_H // 2, pair, 0)
    plsc.subcore_barrier()
    pltpu.sync_copy(acc_sh.at[pl.ds(row0, ROWS_PER_SUB)],
                    out_hbm.at[c, pl.ds(row0, ROWS_PER_SUB)])


@functools.cache
def _sc_scatter():
    return functools.partial(
        pl.kernel,
        mesh=plsc.VectorSubcoreMesh(core_axis_name="c", subcore_axis_name="s"),
        out_type=jax.ShapeDtypeStruct((NC, N_PAD, COUT), jnp.float32),
        scratch_types=[
            pltpu.VMEM((NCH_H, K), jnp.int32),
            pltpu.VMEM((NCH_H, K), jnp.int32),
            pltpu.VMEM((K, COUT), jnp.float32),
            pltpu.VMEM((K, COUT), jnp.float32),
            pltpu.VMEM_SHARED((N_PAD, COUT), jnp.float32),
            pltpu.SemaphoreType.DMA,
            pltpu.SemaphoreType.DMA,
        ],
    )(_sc_body)


def _tc_final_body(x_ref, wr_ref, b_ref, p_ref, o_ref):
    acc = jnp.dot(x_ref[...], wr_ref[...], preferred_element_type=jnp.float32)
    acc = acc + p_ref[0] + p_ref[1] + b_ref[...]
    o_ref[...] = jnp.maximum(acc, 0.0)


_tc_final = pl.pallas_call(
    _tc_final_body,
    grid=(NB,),
    in_specs=[
        pl.BlockSpec((BNF, CIN), lambda i: (i, 0)),
        pl.BlockSpec((CIN, COUT), lambda i: (0, 0)),
        pl.BlockSpec((1, COUT), lambda i: (0, 0)),
        pl.BlockSpec((NC, BNF, COUT), lambda i: (0, i, 0)),
    ],
    out_specs=pl.BlockSpec((BNF, COUT), lambda i: (i, 0)),
    out_shape=jax.ShapeDtypeStruct((N, COUT), jnp.float32),
)


def kernel(x, edge_index, edge_type, W, W_root, b):
    nbt = N // BN
    src = edge_index[0].astype(jnp.int32).reshape(nbt, EB2, 128)
    et = edge_type.astype(jnp.int32).reshape(nbt, EB2, 128)
    xw, fidx = _tc_transform(x, W, src, et)
    npad = EPAD - E
    pad_i = jnp.arange(npad, dtype=jnp.int32)
    fd = jnp.concatenate(
        [fidx.reshape(E), (pad_i * 64) % (N * R),
         edge_index[1].astype(jnp.int32), N + (pad_i % NTRASH)]
    ).reshape(2, NW, NCH, K)
    partials = _sc_scatter()(fd, xw.reshape(N * R, COUT))
    return _tc_final(x, W_root, b.reshape(1, COUT), partials)
